# baseline (speedup 1.0000x reference)
import numpy as np
import jax
import jax.numpy as jnp
from jax import lax
from jax.experimental import pallas as pl
from jax.experimental.pallas import tpu as pltpu
from jax.experimental.pallas import tpu_sc as plsc

_B, _T, _C = 16, 2048, 512
_N = _B * _T

_NW = 32
_RPW = _N // _NW
_K = 16
_NCH = _RPW // _K


def _static_plan():
    rng = np.random.RandomState(0)
    min_len_seg, max_len_seg = 19, 32
    max_num_seg = _T // min_len_seg + 1
    n = _B * max_num_seg
    indices = np.broadcast_to(
        np.arange(max_len_seg * 2)[None, :], (n, max_len_seg * 2))
    scales = rng.rand(n) + 0.5
    idx_scaled = indices / scales[:, None]
    idx_scaled_fl = np.floor(idx_scaled)
    lambda_ = idx_scaled - idx_scaled_fl
    len_seg = rng.randint(min_len_seg, max_len_seg, size=(n, 1))
    idx_mask = idx_scaled_fl < (len_seg - 1)
    offset = np.cumsum(len_seg.reshape(_B, -1), axis=-1)
    offset = np.pad(offset[:, :-1], ((0, 0), (1, 0)),
                    constant_values=0).reshape(-1, 1)
    idx_scaled_org = idx_scaled_fl + offset
    idx_mask_org = idx_scaled_org < (_T - 1)
    m = idx_mask & idx_mask_org
    counts = m.sum(axis=-1).reshape(_B, -1).sum(axis=-1)
    i1 = np.repeat(np.arange(_B), counts)
    i2 = idx_scaled_org[m].astype(np.int64)
    lam = lambda_[m]
    starts = np.concatenate([[0], np.cumsum(counts)[:-1]])
    pos = np.arange(i1.shape[0]) - starts[i1]
    keep = pos < _T
    i1, i2, lam, pos = i1[keep], i2[keep], lam[keep], pos[keep]

    flat = i1 * _T + pos
    g = np.zeros(_N, np.int64)
    g[flat] = i1 * _T + i2
    gpair = np.stack([g, g + 1], axis=1).reshape(-1).astype(np.int32)
    wv = np.zeros((_N, 32), np.float32)
    wv[flat, :16] = (1.0 - lam)[:, None]
    wv[flat, 16:] = lam[:, None]
    return gpair, wv


_GPAIR, _WV = _static_plan()


def _sc_body(x_hbm, gp_hbm, wv_hbm, out_hbm,
             idx0, idx1, wv0, wv1, rows0, rows1, ob0, ob1,
             gs0, gs1, os0, os1):
    idx = (idx0, idx1)
    wvb = (wv0, wv1)
    rows = (rows0, rows1)
    outb = (ob0, ob1)
    gs = (gs0, gs1)
    osem = (os0, os1)
    wid = lax.axis_index("s") * 2 + lax.axis_index("c")
    row0 = wid * _RPW

    def start(c, b):
        base = row0 + c * _K
        pltpu.sync_copy(gp_hbm.at[pl.ds(2 * base, 2 * _K)], idx[b])
        pltpu.sync_copy(wv_hbm.at[pl.ds(base, _K)], wvb[b])
        pltpu.async_copy(x_hbm.at[idx[b]], rows[b], gs[b])

    start(0, 0)
    start(1, 1)

    def iter_fn(g, carry):
        for b in range(2):
            c = 2 * g + b
            base = row0 + c * _K
            pltpu.make_async_copy(x_hbm.at[idx[b]], rows[b], gs[b]).wait()

            @pl.when(g > 0)
            def _wait_out():
                pltpu.make_async_copy(
                    outb[b], out_hbm.at[pl.ds(base, _K)], osem[b]).wait()

            for r in range(_K):
                w0 = wvb[b][r, pl.ds(0, 16)]
                w1 = wvb[b][r, pl.ds(16, 16)]
                for j in range(_C // 16):
                    av = rows[b][2 * r, pl.ds(j * 16, 16)]
                    bv = rows[b][2 * r + 1, pl.ds(j * 16, 16)]
                    outb[b][r, pl.ds(j * 16, 16)] = w0 * av + w1 * bv
            pltpu.async_copy(outb[b], out_hbm.at[pl.ds(base, _K)], osem[b])

            @pl.when(c + 2 < _NCH)
            def _prefetch():
                start(c + 2, b)
        return carry

    lax.fori_loop(0, _NCH // 2, iter_fn, 0)
    for b in range(2):
        pltpu.make_async_copy(
            outb[b], out_hbm.at[pl.ds(row0, _K)], osem[b]).wait()


def kernel(x):
    xf = x.reshape(_N, _C)
    gp = jnp.asarray(_GPAIR)
    wv = jnp.asarray(_WV)
    mesh = plsc.VectorSubcoreMesh(core_axis_name="c", subcore_axis_name="s")
    f = pl.kernel(
        _sc_body,
        out_type=jax.ShapeDtypeStruct((_N, _C), jnp.float32),
        mesh=mesh,
        scratch_types=[
            pltpu.VMEM((2 * _K,), jnp.int32),
            pltpu.VMEM((2 * _K,), jnp.int32),
            pltpu.VMEM((_K, 32), jnp.float32),
            pltpu.VMEM((_K, 32), jnp.float32),
            pltpu.VMEM((2 * _K, _C), jnp.float32),
            pltpu.VMEM((2 * _K, _C), jnp.float32),
            pltpu.VMEM((_K, _C), jnp.float32),
            pltpu.VMEM((_K, _C), jnp.float32),
            pltpu.SemaphoreType.DMA,
            pltpu.SemaphoreType.DMA,
            pltpu.SemaphoreType.DMA,
            pltpu.SemaphoreType.DMA,
        ],
    )
    out = f(xf, gp, wv)
    return out.reshape(_B, _T, _C)


# speedup vs baseline: 1.2099x; 1.2099x over previous
import numpy as np
import jax
import jax.numpy as jnp
from jax import lax
from jax.experimental import pallas as pl
from jax.experimental.pallas import tpu as pltpu
from jax.experimental.pallas import tpu_sc as plsc

_B, _T, _C = 16, 2048, 512
_N = _B * _T

_NW = 32
_RPW = _N // _NW
_K = 16
_NCH = _RPW // _K
_NBUF = 4


def _static_plan():
    rng = np.random.RandomState(0)
    min_len_seg, max_len_seg = 19, 32
    max_num_seg = _T // min_len_seg + 1
    n = _B * max_num_seg
    indices = np.broadcast_to(
        np.arange(max_len_seg * 2)[None, :], (n, max_len_seg * 2))
    scales = rng.rand(n) + 0.5
    idx_scaled = indices / scales[:, None]
    idx_scaled_fl = np.floor(idx_scaled)
    lambda_ = idx_scaled - idx_scaled_fl
    len_seg = rng.randint(min_len_seg, max_len_seg, size=(n, 1))
    idx_mask = idx_scaled_fl < (len_seg - 1)
    offset = np.cumsum(len_seg.reshape(_B, -1), axis=-1)
    offset = np.pad(offset[:, :-1], ((0, 0), (1, 0)),
                    constant_values=0).reshape(-1, 1)
    idx_scaled_org = idx_scaled_fl + offset
    idx_mask_org = idx_scaled_org < (_T - 1)
    m = idx_mask & idx_mask_org
    counts = m.sum(axis=-1).reshape(_B, -1).sum(axis=-1)
    i1 = np.repeat(np.arange(_B), counts)
    i2 = idx_scaled_org[m].astype(np.int64)
    lam = lambda_[m]
    starts = np.concatenate([[0], np.cumsum(counts)[:-1]])
    pos = np.arange(i1.shape[0]) - starts[i1]
    keep = pos < _T
    i1, i2, lam, pos = i1[keep], i2[keep], lam[keep], pos[keep]

    flat = i1 * _T + pos
    g = np.zeros(_N, np.int64)
    g[flat] = i1 * _T + i2
    gpair = np.stack([g, g + 1], axis=1).reshape(-1).astype(np.int32)
    wv = np.zeros((_N, 32), np.float32)
    wv[flat, :16] = (1.0 - lam)[:, None]
    wv[flat, 16:] = lam[:, None]
    return gpair, wv


_GPAIR, _WV = _static_plan()


def _sc_body(x_hbm, gp_hbm, wv_hbm, out_hbm,
             gidx_all,
             wv0, wv1, wv2, wv3,
             rows0, rows1, rows2, rows3,
             ob0, ob1, ob2, ob3,
             gs0, gs1, gs2, gs3,
             os0, os1, os2, os3):
    wvb = (wv0, wv1, wv2, wv3)
    rows = (rows0, rows1, rows2, rows3)
    outb = (ob0, ob1, ob2, ob3)
    gs = (gs0, gs1, gs2, gs3)
    osem = (os0, os1, os2, os3)
    wid = lax.axis_index("s") * 2 + lax.axis_index("c")
    row0 = wid * _RPW

    pltpu.sync_copy(gp_hbm.at[pl.ds(2 * row0, 2 * _RPW)], gidx_all)

    def start(c, b):
        pltpu.async_copy(
            x_hbm.at[gidx_all.at[pl.ds(c * 2 * _K, 2 * _K)]], rows[b], gs[b])
        pltpu.async_copy(
            wv_hbm.at[pl.ds(row0 + c * _K, _K)], wvb[b], gs[b])

    for b in range(_NBUF):
        start(b, b)

    def iter_fn(g, carry):
        for b in range(_NBUF):
            c = _NBUF * g + b
            base = row0 + c * _K
            pltpu.make_async_copy(
                x_hbm.at[gidx_all.at[pl.ds(0, 2 * _K)]], rows[b],
                gs[b]).wait()
            pltpu.make_async_copy(
                wv_hbm.at[pl.ds(0, _K)], wvb[b], gs[b]).wait()

            @pl.when(g > 0)
            def _wait_out():
                pltpu.make_async_copy(
                    outb[b], out_hbm.at[pl.ds(base, _K)], osem[b]).wait()

            def rowfn(r, c2):
                w0 = wvb[b][r, pl.ds(0, 16)]
                w1 = wvb[b][r, pl.ds(16, 16)]
                for j in range(_C // 16):
                    av = rows[b][2 * r, pl.ds(j * 16, 16)]
                    bv = rows[b][2 * r + 1, pl.ds(j * 16, 16)]
                    outb[b][r, pl.ds(j * 16, 16)] = w0 * av + w1 * bv
                return c2

            lax.fori_loop(0, _K, rowfn, 0)
            pltpu.async_copy(outb[b], out_hbm.at[pl.ds(base, _K)], osem[b])

            @pl.when(c + _NBUF < _NCH)
            def _prefetch():
                start(c + _NBUF, b)
        return carry

    lax.fori_loop(0, _NCH // _NBUF, iter_fn, 0)
    for b in range(_NBUF):
        pltpu.make_async_copy(
            outb[b], out_hbm.at[pl.ds(row0, _K)], osem[b]).wait()


def kernel(x):
    xf = x.reshape(_N, _C)
    gp = jnp.asarray(_GPAIR)
    wv = jnp.asarray(_WV)
    mesh = plsc.VectorSubcoreMesh(core_axis_name="c", subcore_axis_name="s")
    f = pl.kernel(
        _sc_body,
        out_type=jax.ShapeDtypeStruct((_N, _C), jnp.float32),
        mesh=mesh,
        scratch_types=(
            [pltpu.VMEM((2 * _RPW,), jnp.int32)]
            + [pltpu.VMEM((_K, 32), jnp.float32) for _ in range(_NBUF)]
            + [pltpu.VMEM((2 * _K, _C), jnp.float32) for _ in range(_NBUF)]
            + [pltpu.VMEM((_K, _C), jnp.float32) for _ in range(_NBUF)]
            + [pltpu.SemaphoreType.DMA for _ in range(2 * _NBUF)]
        ),
    )
    out = f(xf, gp, wv)
    return out.reshape(_B, _T, _C)


# speedup vs baseline: 1.6505x; 1.3641x over previous
import numpy as np
import jax
import jax.numpy as jnp
from jax import lax
from jax.experimental import pallas as pl
from jax.experimental.pallas import tpu as pltpu
from jax.experimental.pallas import tpu_sc as plsc

_B, _T, _C = 16, 2048, 512
_N = _B * _T

_SB = 8
_NB = _B - _SB
_NSC = _SB * _T

_NW = 32
_RPW = _NSC // _NW
_K = 16
_NCH = _RPW // _K
_NBUF = 4


def _static_plan():
    rng = np.random.RandomState(0)
    min_len_seg, max_len_seg = 19, 32
    max_num_seg = _T // min_len_seg + 1
    n = _B * max_num_seg
    indices = np.broadcast_to(
        np.arange(max_len_seg * 2)[None, :], (n, max_len_seg * 2))
    scales = rng.rand(n) + 0.5
    idx_scaled = indices / scales[:, None]
    idx_scaled_fl = np.floor(idx_scaled)
    lambda_ = idx_scaled - idx_scaled_fl
    len_seg = rng.randint(min_len_seg, max_len_seg, size=(n, 1))
    idx_mask = idx_scaled_fl < (len_seg - 1)
    offset = np.cumsum(len_seg.reshape(_B, -1), axis=-1)
    offset = np.pad(offset[:, :-1], ((0, 0), (1, 0)),
                    constant_values=0).reshape(-1, 1)
    idx_scaled_org = idx_scaled_fl + offset
    idx_mask_org = idx_scaled_org < (_T - 1)
    m = idx_mask & idx_mask_org
    counts = m.sum(axis=-1).reshape(_B, -1).sum(axis=-1)
    i1 = np.repeat(np.arange(_B), counts)
    i2 = idx_scaled_org[m].astype(np.int64)
    lam = lambda_[m]
    starts = np.concatenate([[0], np.cumsum(counts)[:-1]])
    pos = np.arange(i1.shape[0]) - starts[i1]
    keep = pos < _T
    i1, i2, lam, pos = i1[keep], i2[keep], lam[keep], pos[keep]

    flat = i1 * _T + pos
    g = np.zeros(_N, np.int64)
    g[flat] = i1 * _T + i2
    gsc = g[:_NSC]
    gpair = np.stack([gsc, gsc + 1], axis=1).reshape(-1).astype(np.int32)
    wv = np.zeros((_NSC, 32), np.float32)
    sel = flat < _NSC
    wv[flat[sel], :16] = (1.0 - lam[sel])[:, None]
    wv[flat[sel], 16:] = lam[sel][:, None]

    idx_t = np.zeros((_NB, _T), np.int32)
    w0_t = np.zeros((_NB, _T), np.float32)
    w1_t = np.zeros((_NB, _T), np.float32)
    sel = i1 >= _SB
    idx_t[i1[sel] - _SB, pos[sel]] = i2[sel]
    w0_t[i1[sel] - _SB, pos[sel]] = 1.0 - lam[sel]
    w1_t[i1[sel] - _SB, pos[sel]] = lam[sel]
    return (gpair, wv,
            idx_t.reshape(_NB, 1, _T),
            w0_t.reshape(_NB, 1, _T),
            w1_t.reshape(_NB, 1, _T))


_GPAIR, _WV, _IDXT, _W0T, _W1T = _static_plan()


def _sc_body(x_hbm, gp_hbm, wv_hbm, out_hbm,
             gidx_all,
             wv0, wv1, wv2, wv3,
             rows0, rows1, rows2, rows3,
             ob0, ob1, ob2, ob3,
             gs0, gs1, gs2, gs3,
             os0, os1, os2, os3):
    wvb = (wv0, wv1, wv2, wv3)
    rows = (rows0, rows1, rows2, rows3)
    outb = (ob0, ob1, ob2, ob3)
    gs = (gs0, gs1, gs2, gs3)
    osem = (os0, os1, os2, os3)
    wid = lax.axis_index("s") * 2 + lax.axis_index("c")
    row0 = wid * _RPW

    pltpu.sync_copy(gp_hbm.at[pl.ds(2 * row0, 2 * _RPW)], gidx_all)

    def start(c, b):
        pltpu.async_copy(
            x_hbm.at[gidx_all.at[pl.ds(c * 2 * _K, 2 * _K)]], rows[b], gs[b])
        pltpu.async_copy(
            wv_hbm.at[pl.ds(row0 + c * _K, _K)], wvb[b], gs[b])

    for b in range(_NBUF):
        start(b, b)

    def iter_fn(g, carry):
        for b in range(_NBUF):
            c = _NBUF * g + b
            base = row0 + c * _K
            pltpu.make_async_copy(
                x_hbm.at[gidx_all.at[pl.ds(0, 2 * _K)]], rows[b],
                gs[b]).wait()
            pltpu.make_async_copy(
                wv_hbm.at[pl.ds(0, _K)], wvb[b], gs[b]).wait()

            @pl.when(g > 0)
            def _wait_out():
                pltpu.make_async_copy(
                    outb[b], out_hbm.at[pl.ds(base, _K)], osem[b]).wait()

            def rowfn(r, c2):
                w0 = wvb[b][r, pl.ds(0, 16)]
                w1 = wvb[b][r, pl.ds(16, 16)]
                for j in range(_C // 16):
                    av = rows[b][2 * r, pl.ds(j * 16, 16)]
                    bv = rows[b][2 * r + 1, pl.ds(j * 16, 16)]
                    outb[b][r, pl.ds(j * 16, 16)] = w0 * av + w1 * bv
                return c2

            lax.fori_loop(0, _K, rowfn, 0)
            pltpu.async_copy(outb[b], out_hbm.at[pl.ds(base, _K)], osem[b])

            @pl.when(c + _NBUF < _NCH)
            def _prefetch():
                start(c + _NBUF, b)
        return carry

    lax.fori_loop(0, _NCH // _NBUF, iter_fn, 0)
    for b in range(_NBUF):
        pltpu.make_async_copy(
            outb[b], out_hbm.at[pl.ds(row0, _K)], osem[b]).wait()


def _tc_body(x_ref, idx_ref, w0_ref, w1_ref, o_ref):
    rows = x_ref[0]
    idx = idx_ref[0, 0]
    w0 = w0_ref[0, 0]
    w1 = w1_ref[0, 0]
    iota = lax.broadcasted_iota(jnp.int32, (_T, _T), 1)
    s = (jnp.where(iota == idx[:, None], w0[:, None], 0.0)
         + jnp.where(iota == idx[:, None] + 1, w1[:, None], 0.0))
    o_ref[0] = jnp.dot(s, rows, preferred_element_type=jnp.float32)


def kernel(x):
    xf = x.reshape(_N, _C)
    gp = jnp.asarray(_GPAIR)
    wv = jnp.asarray(_WV)
    mesh = plsc.VectorSubcoreMesh(core_axis_name="c", subcore_axis_name="s")
    f = pl.kernel(
        _sc_body,
        out_type=jax.ShapeDtypeStruct((_NSC, _C), jnp.float32),
        mesh=mesh,
        scratch_types=(
            [pltpu.VMEM((2 * _RPW,), jnp.int32)]
            + [pltpu.VMEM((_K, 32), jnp.float32) for _ in range(_NBUF)]
            + [pltpu.VMEM((2 * _K, _C), jnp.float32) for _ in range(_NBUF)]
            + [pltpu.VMEM((_K, _C), jnp.float32) for _ in range(_NBUF)]
            + [pltpu.SemaphoreType.DMA for _ in range(2 * _NBUF)]
        ),
    )
    out_sc = f(xf, gp, wv).reshape(_SB, _T, _C)

    out_tc = pl.pallas_call(
        _tc_body,
        grid=(_NB,),
        in_specs=[
            pl.BlockSpec((1, _T, _C), lambda b: (b, 0, 0)),
            pl.BlockSpec((1, 1, _T), lambda b: (b, 0, 0)),
            pl.BlockSpec((1, 1, _T), lambda b: (b, 0, 0)),
            pl.BlockSpec((1, 1, _T), lambda b: (b, 0, 0)),
        ],
        out_specs=pl.BlockSpec((1, _T, _C), lambda b: (b, 0, 0)),
        out_shape=jax.ShapeDtypeStruct((_NB, _T, _C), jnp.float32),
    )(x[_SB:], jnp.asarray(_IDXT), jnp.asarray(_W0T), jnp.asarray(_W1T))

    return jnp.concatenate([out_sc, out_tc], axis=0)
